# Initial kernel scaffold; baseline (speedup 1.0000x reference)
import jax
import jax.numpy as jnp
from jax.experimental import pallas as pl


def kernel(h, z_coords, batch, W1, b1, Wz1, bz1, Wz2, bz2, W2, b2):
    raise NotImplementedError("write your pallas kernel here")


# speedup vs baseline: 2.9049x; 2.9049x over previous
import functools

import jax
import jax.numpy as jnp
from jax import lax
from jax.experimental import pallas as pl
from jax.experimental.pallas import tpu as pltpu
from jax.experimental.pallas import tpu_sc as plsc

_N = 100000
_H = 256
_HO = 128
_ZD = 32
_NSEG = 1000

_R = 2000
_NB = _N // _R

_NW = 32
_KC = 25
_NPAD = _NW * _KC * 128
_ACC = 1024


def _tc_body(h_ref, z_ref, w1_ref, b1_ref, wz1_ref, bz1_ref, wz2_ref,
             bz2_ref, wh_ref, wzv_ref, v_ref):
    hb = h_ref[...]
    hh = jnp.dot(hb, w1_ref[...], preferred_element_type=jnp.float32)
    hh = hh + b1_ref[...]
    hh = hh * jax.nn.sigmoid(hh)
    vh = jnp.sum(hh * wh_ref[...], axis=1)

    zb = z_ref[...]
    t = zb * wz1_ref[...] + bz1_ref[...]
    t = t * jax.nn.sigmoid(t)
    u = jnp.dot(t, wz2_ref[...], preferred_element_type=jnp.float32)
    u = u + bz2_ref[...]
    u = u * jax.nn.sigmoid(u)
    vz = jnp.sum(u * wzv_ref[...], axis=1)

    v_ref[...] = (vh + vz).reshape(1, 1, _R)


def _compute_v(h, z_coords, w1t, b1r, wz1r, bz1r, wz2t, bz2r, wh, wzv):
    return pl.pallas_call(
        _tc_body,
        grid=(_NB,),
        in_specs=[
            pl.BlockSpec((_R, _H), lambda i: (i, 0)),
            pl.BlockSpec((_R, 1), lambda i: (i, 0)),
            pl.BlockSpec((_H, _HO), lambda i: (0, 0)),
            pl.BlockSpec((1, _HO), lambda i: (0, 0)),
            pl.BlockSpec((1, 16), lambda i: (0, 0)),
            pl.BlockSpec((1, 16), lambda i: (0, 0)),
            pl.BlockSpec((16, _ZD), lambda i: (0, 0)),
            pl.BlockSpec((1, _ZD), lambda i: (0, 0)),
            pl.BlockSpec((1, _HO), lambda i: (0, 0)),
            pl.BlockSpec((1, _ZD), lambda i: (0, 0)),
        ],
        out_specs=pl.BlockSpec((1, 1, _R), lambda i: (i, 0, 0)),
        out_shape=jax.ShapeDtypeStruct((_NB, 1, _R), jnp.float32),
    )(h, z_coords, w1t, b1r, wz1r, bz1r, wz2t, bz2r, wh, wzv)


def _sc_segment_sum(idx3, val3):
    mesh = plsc.VectorSubcoreMesh(core_axis_name="c", subcore_axis_name="s")

    @functools.partial(
        pl.kernel,
        out_type=jax.ShapeDtypeStruct((2, _ACC), jnp.float32),
        mesh=mesh,
        scratch_types=[
            pltpu.VMEM((_KC, 128), jnp.int32),
            pltpu.VMEM((_KC, 128), jnp.float32),
            pltpu.VMEM((_ACC,), jnp.float32),
            pltpu.VMEM_SHARED((_ACC,), jnp.float32),
        ],
    )
    def k(idx_hbm, val_hbm, out_hbm, idx_v, val_v, zbuf, acc_sh):
        c = lax.axis_index("c")
        s = lax.axis_index("s")
        w = s * 2 + c

        @pl.when(s == 0)
        def _zero():
            for kk in range(_ACC // 16):
                zbuf[pl.ds(kk * 16, 16)] = jnp.zeros((16,), jnp.float32)
            pltpu.sync_copy(zbuf, acc_sh)

        plsc.subcore_barrier()

        pltpu.sync_copy(idx_hbm.at[w], idx_v)
        pltpu.sync_copy(val_hbm.at[w], val_v)
        for j in range(_KC):
            pltpu.sync_copy(val_v.at[j], acc_sh.at[idx_v.at[j]], add=True)

        plsc.subcore_barrier()

        @pl.when(s == 0)
        def _out():
            pltpu.sync_copy(acc_sh, out_hbm.at[c])

    return k(idx3, val3)


def kernel(h, z_coords, batch, W1, b1, Wz1, bz1, Wz2, bz2, W2, b2):
    w1t = W1.T
    b1r = b1.reshape(1, _HO)
    wz1r = Wz1.reshape(1, 16)
    bz1r = bz1.reshape(1, 16)
    wz2t = Wz2.T
    bz2r = bz2.reshape(1, _ZD)
    wh = W2[:, :_HO].reshape(1, _HO)
    wzv = W2[:, _HO:].reshape(1, _ZD)

    v = _compute_v(h, z_coords, w1t, b1r, wz1r, bz1r, wz2t, bz2r, wh, wzv)

    pad = _NPAD - _N
    idx = batch.astype(jnp.int32)
    idx_pad = _NSEG + (jnp.arange(pad, dtype=jnp.int32) % (_ACC - _NSEG))
    idx3 = jnp.concatenate([idx, idx_pad]).reshape(_NW, _KC, 128)
    val3 = jnp.concatenate(
        [v.reshape(_N), jnp.zeros((pad,), jnp.float32)]).reshape(_NW, _KC, 128)

    parts = _sc_segment_sum(idx3, val3)
    seg = parts[0, :_NSEG] + parts[1, :_NSEG]
    return (seg + b2[0]).reshape(_NSEG, 1)


# speedup vs baseline: 3.6522x; 1.2572x over previous
import functools

import jax
import jax.numpy as jnp
from jax import lax
from jax.experimental import pallas as pl
from jax.experimental.pallas import tpu as pltpu
from jax.experimental.pallas import tpu_sc as plsc

_N = 100000
_H = 256
_HO = 128
_ZD = 32
_NSEG = 1000

_R = 2000
_NB = _N // _R

_NW = 32
_KC = 25
_NPAD = _NW * _KC * 128
_ACC = 1024


def _tc_body(h_ref, z_ref, w1_ref, b1_ref, wz1_ref, bz1_ref, wz2_ref,
             bz2_ref, wh_ref, wzv_ref, v_ref):
    hb = h_ref[...]
    hh = jnp.dot(hb, w1_ref[...], preferred_element_type=jnp.float32)
    hh = hh + b1_ref[...]
    hh = hh * jax.nn.sigmoid(hh)
    vh = jnp.dot(hh, wh_ref[...], preferred_element_type=jnp.float32)

    zb = z_ref[...]
    t = zb * wz1_ref[...] + bz1_ref[...]
    t = t * jax.nn.sigmoid(t)
    u = jnp.dot(t, wz2_ref[...], preferred_element_type=jnp.float32)
    u = u + bz2_ref[...]
    u = u * jax.nn.sigmoid(u)
    vz = jnp.dot(u, wzv_ref[...], preferred_element_type=jnp.float32)

    v_ref[...] = vh + vz


def _compute_v(h, z_coords, w1t, b1r, wz1r, bz1r, wz2t, bz2r, wh, wzv):
    return pl.pallas_call(
        _tc_body,
        grid=(_NB,),
        in_specs=[
            pl.BlockSpec((_R, _H), lambda i: (i, 0)),
            pl.BlockSpec((_R, 1), lambda i: (i, 0)),
            pl.BlockSpec((_H, _HO), lambda i: (0, 0)),
            pl.BlockSpec((1, _HO), lambda i: (0, 0)),
            pl.BlockSpec((1, 16), lambda i: (0, 0)),
            pl.BlockSpec((1, 16), lambda i: (0, 0)),
            pl.BlockSpec((16, _ZD), lambda i: (0, 0)),
            pl.BlockSpec((1, _ZD), lambda i: (0, 0)),
            pl.BlockSpec((_HO, 1), lambda i: (0, 0)),
            pl.BlockSpec((_ZD, 1), lambda i: (0, 0)),
        ],
        out_specs=pl.BlockSpec((_R, 1), lambda i: (i, 0)),
        out_shape=jax.ShapeDtypeStruct((_N, 1), jnp.float32),
    )(h, z_coords, w1t, b1r, wz1r, bz1r, wz2t, bz2r, wh, wzv)


def _sc_segment_sum(idx3, val3):
    mesh = plsc.VectorSubcoreMesh(core_axis_name="c", subcore_axis_name="s")

    @functools.partial(
        pl.kernel,
        out_type=jax.ShapeDtypeStruct((2, _ACC), jnp.float32),
        mesh=mesh,
        scratch_types=[
            pltpu.VMEM((_KC, 128), jnp.int32),
            pltpu.VMEM((_KC, 128), jnp.float32),
            pltpu.VMEM((_ACC,), jnp.float32),
            pltpu.VMEM_SHARED((_ACC,), jnp.float32),
        ],
    )
    def k(idx_hbm, val_hbm, out_hbm, idx_v, val_v, zbuf, acc_sh):
        c = lax.axis_index("c")
        s = lax.axis_index("s")
        w = s * 2 + c

        @pl.when(s == 0)
        def _zero():
            for kk in range(_ACC // 16):
                zbuf[pl.ds(kk * 16, 16)] = jnp.zeros((16,), jnp.float32)
            pltpu.sync_copy(zbuf, acc_sh)

        plsc.subcore_barrier()

        pltpu.sync_copy(idx_hbm.at[w], idx_v)
        pltpu.sync_copy(val_hbm.at[w], val_v)
        for j in range(_KC):
            pltpu.sync_copy(val_v.at[j], acc_sh.at[idx_v.at[j]], add=True)

        plsc.subcore_barrier()

        @pl.when(s == 0)
        def _out():
            pltpu.sync_copy(acc_sh, out_hbm.at[c])

    return k(idx3, val3)


def kernel(h, z_coords, batch, W1, b1, Wz1, bz1, Wz2, bz2, W2, b2):
    w1t = W1.T
    b1r = b1.reshape(1, _HO)
    wz1r = Wz1.reshape(1, 16)
    bz1r = bz1.reshape(1, 16)
    wz2t = Wz2.T
    bz2r = bz2.reshape(1, _ZD)
    wh = W2[:, :_HO].reshape(_HO, 1)
    wzv = W2[:, _HO:].reshape(_ZD, 1)

    v = _compute_v(h, z_coords, w1t, b1r, wz1r, bz1r, wz2t, bz2r, wh, wzv)

    pad = _NPAD - _N
    idx = batch.astype(jnp.int32)
    idx_pad = _NSEG + (jnp.arange(pad, dtype=jnp.int32) % (_ACC - _NSEG))
    idx3 = jnp.concatenate([idx, idx_pad]).reshape(_NW, _KC, 128)
    val3 = jnp.concatenate(
        [v.reshape(_N), jnp.zeros((pad,), jnp.float32)]).reshape(_NW, _KC, 128)

    parts = _sc_segment_sum(idx3, val3)
    seg = parts[0, :_NSEG] + parts[1, :_NSEG]
    return (seg + b2[0]).reshape(_NSEG, 1)


# speedup vs baseline: 5.7284x; 1.5685x over previous
import functools

import jax
import jax.numpy as jnp
from jax import lax
from jax.experimental import pallas as pl
from jax.experimental.pallas import tpu as pltpu
from jax.experimental.pallas import tpu_sc as plsc

_N = 100000
_H = 256
_HO = 128
_ZD = 32
_NSEG = 1000

_R = 2048
_NB = 50
_NPAD = _NB * _R
_LASTH = 48

_NW = 32
_KW = _NPAD // _NW // 128
_ROWS = _NPAD // 128
_ACC = 1024


def _tc_body(h_ref, z_ref, w1_ref, b1_ref, wz1_ref, bz1_ref, wz2_ref,
             bz2_ref, wh_ref, wzv_ref, v_ref):
    hb = h_ref[...]
    hhT = lax.dot_general(w1_ref[...], hb, (((1,), (1,)), ((), ())),
                          preferred_element_type=jnp.float32)
    hhT = hhT + b1_ref[...]
    hhT = hhT * jax.nn.sigmoid(hhT)
    vh = jnp.dot(wh_ref[...], hhT, preferred_element_type=jnp.float32)

    zr = z_ref[...].reshape(1, _R)
    t = jnp.dot(wz1_ref[...], zr, preferred_element_type=jnp.float32)
    t = t + bz1_ref[...]
    t = t * jax.nn.sigmoid(t)
    u = jnp.dot(wz2_ref[...], t, preferred_element_type=jnp.float32)
    u = u + bz2_ref[...]
    u = u * jax.nn.sigmoid(u)
    vz = jnp.dot(wzv_ref[...], u, preferred_element_type=jnp.float32)

    v_ref[...] = (vh + vz).reshape(1, 16, 128)


def _compute_v(h, z2, w1, b1c, wz1c, bz1c, wz2, bz2c, whr, wzvr):
    return pl.pallas_call(
        _tc_body,
        grid=(_NB,),
        in_specs=[
            pl.BlockSpec((_R, _H), lambda i: (jnp.minimum(i, _LASTH), 0)),
            pl.BlockSpec((16, 128), lambda i: (i, 0)),
            pl.BlockSpec((_HO, _H), lambda i: (0, 0)),
            pl.BlockSpec((_HO, 1), lambda i: (0, 0)),
            pl.BlockSpec((16, 1), lambda i: (0, 0)),
            pl.BlockSpec((16, 1), lambda i: (0, 0)),
            pl.BlockSpec((_ZD, 16), lambda i: (0, 0)),
            pl.BlockSpec((_ZD, 1), lambda i: (0, 0)),
            pl.BlockSpec((1, _HO), lambda i: (0, 0)),
            pl.BlockSpec((1, _ZD), lambda i: (0, 0)),
        ],
        out_specs=pl.BlockSpec((1, 16, 128), lambda i: (i, 0, 0)),
        out_shape=jax.ShapeDtypeStruct((_NB, 16, 128), jnp.float32),
    )(h, z2, w1, b1c, wz1c, bz1c, wz2, bz2c, whr, wzvr)


def _sc_segment_sum(idx3, val3):
    mesh = plsc.VectorSubcoreMesh(core_axis_name="c", subcore_axis_name="s")

    @functools.partial(
        pl.kernel,
        out_type=jax.ShapeDtypeStruct((2, _ACC), jnp.float32),
        mesh=mesh,
        scratch_types=[
            pltpu.VMEM((_KW, 128), jnp.int32),
            pltpu.VMEM((_KW, 128), jnp.float32),
            pltpu.VMEM((_ACC,), jnp.float32),
            pltpu.VMEM_SHARED((_ACC,), jnp.float32),
        ],
    )
    def k(idx_hbm, val_hbm, out_hbm, idx_v, val_v, zbuf, acc_sh):
        c = lax.axis_index("c")
        s = lax.axis_index("s")
        w = s * 2 + c

        @pl.when(s == 0)
        def _zero():
            for kk in range(_ACC // 16):
                zbuf[pl.ds(kk * 16, 16)] = jnp.zeros((16,), jnp.float32)
            pltpu.sync_copy(zbuf, acc_sh)

        plsc.subcore_barrier()

        pltpu.sync_copy(idx_hbm.at[w], idx_v)
        pltpu.sync_copy(val_hbm.at[w], val_v)
        for j in range(_KW):
            pltpu.sync_copy(val_v.at[j], acc_sh.at[idx_v.at[j]], add=True)

        plsc.subcore_barrier()

        @pl.when(s == 0)
        def _out():
            pltpu.sync_copy(acc_sh, out_hbm.at[c])

    return k(idx3, val3)


def kernel(h, z_coords, batch, W1, b1, Wz1, bz1, Wz2, bz2, W2, b2):
    pad = _NPAD - _N
    b1c = b1.reshape(_HO, 1)
    wz1c = Wz1.reshape(16, 1)
    bz1c = bz1.reshape(16, 1)
    bz2c = bz2.reshape(_ZD, 1)
    whr = W2[:, :_HO].reshape(1, _HO)
    wzvr = W2[:, _HO:].reshape(1, _ZD)

    z2 = jnp.concatenate(
        [z_coords.reshape(_N), jnp.zeros((pad,), jnp.float32)]
    ).reshape(_ROWS, 128)

    v = _compute_v(h, z2, W1, b1c, wz1c, bz1c, Wz2, bz2c, whr, wzvr)

    idx = batch.astype(jnp.int32)
    idx_pad = _NSEG + (jnp.arange(pad, dtype=jnp.int32) % (_ACC - _NSEG))
    idx3 = jnp.concatenate([idx, idx_pad]).reshape(_NW, _KW, 128)

    parts = _sc_segment_sum(idx3, v.reshape(_NW, _KW, 128))
    seg = parts[0, :_NSEG] + parts[1, :_NSEG]
    return (seg + b2[0]).reshape(_NSEG, 1)


# speedup vs baseline: 5.7315x; 1.0005x over previous
import functools

import jax
import jax.numpy as jnp
from jax import lax
from jax.experimental import pallas as pl
from jax.experimental.pallas import tpu as pltpu
from jax.experimental.pallas import tpu_sc as plsc

_N = 100000
_H = 256
_HO = 128
_ZD = 32
_NSEG = 1000

_R = 2048
_NB = 50
_NPAD = _NB * _R
_LASTH = 48

_NW = 32
_KW = _NPAD // _NW // 128
_ROWS = _NPAD // 128
_ACC = 1024


def _tc_body(h_ref, z_ref, w1_ref, b1_ref, wz1_ref, bz1_ref, wz2_ref,
             bz2_ref, wh_ref, wzv_ref, v_ref):
    hb = h_ref[...]
    hhT = lax.dot_general(w1_ref[...], hb, (((1,), (1,)), ((), ())),
                          preferred_element_type=jnp.float32,
                          precision=lax.Precision.DEFAULT)
    hhT = hhT + b1_ref[...]
    hhT = hhT * jax.nn.sigmoid(hhT)
    vh = jnp.dot(wh_ref[...], hhT, preferred_element_type=jnp.float32,
                 precision=lax.Precision.DEFAULT)

    zr = z_ref[...].reshape(1, _R)
    t = jnp.dot(wz1_ref[...], zr, preferred_element_type=jnp.float32)
    t = t + bz1_ref[...]
    t = t * jax.nn.sigmoid(t)
    u = jnp.dot(wz2_ref[...], t, preferred_element_type=jnp.float32)
    u = u + bz2_ref[...]
    u = u * jax.nn.sigmoid(u)
    vz = jnp.dot(wzv_ref[...], u, preferred_element_type=jnp.float32)

    v_ref[...] = (vh + vz).reshape(1, 16, 128)


def _compute_v(h, z2, w1, b1c, wz1c, bz1c, wz2, bz2c, whr, wzvr):
    return pl.pallas_call(
        _tc_body,
        grid=(_NB,),
        in_specs=[
            pl.BlockSpec((_R, _H), lambda i: (jnp.minimum(i, _LASTH), 0)),
            pl.BlockSpec((16, 128), lambda i: (i, 0)),
            pl.BlockSpec((_HO, _H), lambda i: (0, 0)),
            pl.BlockSpec((_HO, 1), lambda i: (0, 0)),
            pl.BlockSpec((16, 1), lambda i: (0, 0)),
            pl.BlockSpec((16, 1), lambda i: (0, 0)),
            pl.BlockSpec((_ZD, 16), lambda i: (0, 0)),
            pl.BlockSpec((_ZD, 1), lambda i: (0, 0)),
            pl.BlockSpec((1, _HO), lambda i: (0, 0)),
            pl.BlockSpec((1, _ZD), lambda i: (0, 0)),
        ],
        out_specs=pl.BlockSpec((1, 16, 128), lambda i: (i, 0, 0)),
        out_shape=jax.ShapeDtypeStruct((_NB, 16, 128), jnp.float32),
    )(h, z2, w1, b1c, wz1c, bz1c, wz2, bz2c, whr, wzvr)


def _sc_segment_sum(idx3, val3):
    mesh = plsc.VectorSubcoreMesh(core_axis_name="c", subcore_axis_name="s")

    @functools.partial(
        pl.kernel,
        out_type=jax.ShapeDtypeStruct((2, _ACC), jnp.float32),
        mesh=mesh,
        scratch_types=[
            pltpu.VMEM((_KW, 128), jnp.int32),
            pltpu.VMEM((_KW, 128), jnp.float32),
            pltpu.VMEM((_ACC,), jnp.float32),
            pltpu.VMEM((16, 128), jnp.float32),
            pltpu.VMEM((128,), jnp.float32),
            pltpu.VMEM_SHARED((16 * _ACC,), jnp.float32),
        ],
        compiler_params=pltpu.CompilerParams(needs_layout_passes=False),
    )
    def k(idx_hbm, val_hbm, out_hbm, idx_v, val_v, acc, rbuf, obuf, stage):
        c = lax.axis_index("c")
        s = lax.axis_index("s")
        w = s * 2 + c

        for kk in range(_ACC // 16):
            acc[pl.ds(kk * 16, 16)] = jnp.zeros((16,), jnp.float32)

        pltpu.sync_copy(idx_hbm.at[w], idx_v)
        pltpu.sync_copy(val_hbm.at[w], val_v)
        for j in range(_KW):
            for t in range(8):
                sl = pl.ds(t * 16, 16)
                plsc.addupdate_scatter(acc, [idx_v[j, sl]], val_v[j, sl])

        pltpu.sync_copy(acc, stage.at[pl.ds(s * _ACC, _ACC)])
        plsc.subcore_barrier()

        @pl.when(s < 8)
        def _reduce():
            for r in range(16):
                pltpu.sync_copy(stage.at[pl.ds(r * _ACC + s * 128, 128)],
                                rbuf.at[r])
            for c8 in range(8):
                a = rbuf[0, pl.ds(c8 * 16, 16)]
                for r in range(1, 16):
                    a = a + rbuf[r, pl.ds(c8 * 16, 16)]
                obuf[pl.ds(c8 * 16, 16)] = a
            pltpu.sync_copy(obuf, out_hbm.at[c, pl.ds(s * 128, 128)])

    return k(idx3, val3)


def kernel(h, z_coords, batch, W1, b1, Wz1, bz1, Wz2, bz2, W2, b2):
    pad = _NPAD - _N
    b1c = b1.reshape(_HO, 1)
    wz1c = Wz1.reshape(16, 1)
    bz1c = bz1.reshape(16, 1)
    bz2c = bz2.reshape(_ZD, 1)
    whr = W2[:, :_HO].reshape(1, _HO)
    wzvr = W2[:, _HO:].reshape(1, _ZD)

    z2 = jnp.concatenate(
        [z_coords.reshape(_N), jnp.zeros((pad,), jnp.float32)]
    ).reshape(_ROWS, 128)

    v = _compute_v(h, z2, W1, b1c, wz1c, bz1c, Wz2, bz2c, whr, wzvr)

    idx = batch.astype(jnp.int32)
    idx_pad = _NSEG + (jnp.arange(pad, dtype=jnp.int32) % (_ACC - _NSEG))
    idx3 = jnp.concatenate([idx, idx_pad]).reshape(_NW, _KW, 128)

    parts = _sc_segment_sum(idx3, v.reshape(_NW, _KW, 128))
    seg = parts[0, :_NSEG] + parts[1, :_NSEG]
    return (seg + b2[0]).reshape(_NSEG, 1)


# speedup vs baseline: 6.7939x; 1.1854x over previous
import functools

import jax
import jax.numpy as jnp
from jax import lax
from jax.experimental import pallas as pl
from jax.experimental.pallas import tpu as pltpu
from jax.experimental.pallas import tpu_sc as plsc

_N = 100000
_H = 256
_HO = 128
_ZD = 32
_NSEG = 1000

_R = 4096
_NB = 25
_NPAD = _NB * _R
_RROW = _R // 128

_NW = 32
_KW = _NPAD // _NW // 128
_ROWS = _NPAD // 128
_ACC = 1024


def _tc_body(h_ref, z_ref, w1_ref, b1_ref, wz1_ref, bz1_ref, wz2_ref,
             bz2_ref, wh_ref, wzv_ref, v_ref):
    hb = h_ref[...]
    hhT = lax.dot_general(w1_ref[...], hb, (((1,), (1,)), ((), ())),
                          preferred_element_type=jnp.float32,
                          precision=lax.Precision.DEFAULT)
    hhT = hhT + b1_ref[...]
    hhT = hhT * jax.nn.sigmoid(hhT)
    vh = jnp.dot(wh_ref[...], hhT, preferred_element_type=jnp.float32,
                 precision=lax.Precision.DEFAULT)

    zr = z_ref[...].reshape(1, _R)
    t = jnp.dot(wz1_ref[...], zr, preferred_element_type=jnp.float32)
    t = t + bz1_ref[...]
    t = t * jax.nn.sigmoid(t)
    u = jnp.dot(wz2_ref[...], t, preferred_element_type=jnp.float32)
    u = u + bz2_ref[...]
    u = u * jax.nn.sigmoid(u)
    vz = jnp.dot(wzv_ref[...], u, preferred_element_type=jnp.float32)

    v_ref[...] = (vh + vz).reshape(1, _RROW, 128)


def _compute_v(h, z2, w1, b1c, wz1c, bz1c, wz2, bz2c, whr, wzvr):
    return pl.pallas_call(
        _tc_body,
        grid=(_NB,),
        in_specs=[
            pl.BlockSpec((_R, _H), lambda i: (i, 0)),
            pl.BlockSpec((_RROW, 128), lambda i: (i, 0)),
            pl.BlockSpec((_HO, _H), lambda i: (0, 0)),
            pl.BlockSpec((_HO, 1), lambda i: (0, 0)),
            pl.BlockSpec((16, 1), lambda i: (0, 0)),
            pl.BlockSpec((16, 1), lambda i: (0, 0)),
            pl.BlockSpec((_ZD, 16), lambda i: (0, 0)),
            pl.BlockSpec((_ZD, 1), lambda i: (0, 0)),
            pl.BlockSpec((1, _HO), lambda i: (0, 0)),
            pl.BlockSpec((1, _ZD), lambda i: (0, 0)),
        ],
        out_specs=pl.BlockSpec((1, _RROW, 128), lambda i: (i, 0, 0)),
        out_shape=jax.ShapeDtypeStruct((_NB, _RROW, 128), jnp.float32),
    )(h, z2, w1, b1c, wz1c, bz1c, wz2, bz2c, whr, wzvr)


def _sc_segment_sum(idx3, val3):
    mesh = plsc.VectorSubcoreMesh(core_axis_name="c", subcore_axis_name="s")

    @functools.partial(
        pl.kernel,
        out_type=jax.ShapeDtypeStruct((2, _ACC), jnp.float32),
        mesh=mesh,
        scratch_types=[
            pltpu.VMEM((_KW, 128), jnp.int32),
            pltpu.VMEM((_KW, 128), jnp.float32),
            pltpu.VMEM((_ACC,), jnp.float32),
            pltpu.VMEM((16, 128), jnp.float32),
            pltpu.VMEM((128,), jnp.float32),
            pltpu.VMEM_SHARED((16 * _ACC,), jnp.float32),
        ],
        compiler_params=pltpu.CompilerParams(needs_layout_passes=False),
    )
    def k(idx_hbm, val_hbm, out_hbm, idx_v, val_v, acc, rbuf, obuf, stage):
        c = lax.axis_index("c")
        s = lax.axis_index("s")
        w = s * 2 + c

        for kk in range(_ACC // 16):
            acc[pl.ds(kk * 16, 16)] = jnp.zeros((16,), jnp.float32)

        pltpu.sync_copy(idx_hbm.at[w], idx_v)
        pltpu.sync_copy(val_hbm.at[w], val_v)
        for j in range(_KW):
            for t in range(8):
                sl = pl.ds(t * 16, 16)
                plsc.addupdate_scatter(acc, [idx_v[j, sl]], val_v[j, sl])

        pltpu.sync_copy(acc, stage.at[pl.ds(s * _ACC, _ACC)])
        plsc.subcore_barrier()

        @pl.when(s < 8)
        def _reduce():
            for r in range(16):
                pltpu.sync_copy(stage.at[pl.ds(r * _ACC + s * 128, 128)],
                                rbuf.at[r])
            for c8 in range(8):
                a = rbuf[0, pl.ds(c8 * 16, 16)]
                for r in range(1, 16):
                    a = a + rbuf[r, pl.ds(c8 * 16, 16)]
                obuf[pl.ds(c8 * 16, 16)] = a
            pltpu.sync_copy(obuf, out_hbm.at[c, pl.ds(s * 128, 128)])

    return k(idx3, val3)


def kernel(h, z_coords, batch, W1, b1, Wz1, bz1, Wz2, bz2, W2, b2):
    pad = _NPAD - _N
    b1c = b1.reshape(_HO, 1)
    wz1c = Wz1.reshape(16, 1)
    bz1c = bz1.reshape(16, 1)
    bz2c = bz2.reshape(_ZD, 1)
    whr = W2[:, :_HO].reshape(1, _HO)
    wzvr = W2[:, _HO:].reshape(1, _ZD)

    z2 = jnp.concatenate(
        [z_coords.reshape(_N), jnp.zeros((pad,), jnp.float32)]
    ).reshape(_ROWS, 128)

    v = _compute_v(h, z2, W1, b1c, wz1c, bz1c, Wz2, bz2c, whr, wzvr)

    idx = batch.astype(jnp.int32)
    idx_pad = _NSEG + (jnp.arange(pad, dtype=jnp.int32) % (_ACC - _NSEG))
    idx3 = jnp.concatenate([idx, idx_pad]).reshape(_NW, _KW, 128)

    parts = _sc_segment_sum(idx3, v.reshape(_NW, _KW, 128))
    seg = parts[0, :_NSEG] + parts[1, :_NSEG]
    return (seg + b2[0]).reshape(_NSEG, 1)


# speedup vs baseline: 7.5248x; 1.1076x over previous
import functools

import jax
import jax.numpy as jnp
from jax import lax
from jax.experimental import pallas as pl
from jax.experimental.pallas import tpu as pltpu
from jax.experimental.pallas import tpu_sc as plsc

_N = 100000
_H = 256
_HO = 128
_ZD = 32
_NSEG = 1000

_R = 10240
_NB = 10
_NPAD = _NB * _R
_RROW = _R // 128

_NW = 32
_KW = _NPAD // _NW // 128
_ROWS = _NPAD // 128
_ACC = 1024


def _tc_body(h_ref, z_ref, w1_ref, b1_ref, wz1_ref, bz1_ref, wz2_ref,
             bz2_ref, wh_ref, wzv_ref, v_ref):
    hb = h_ref[...]
    hhT = lax.dot_general(w1_ref[...], hb, (((1,), (1,)), ((), ())),
                          preferred_element_type=jnp.float32,
                          precision=lax.Precision.DEFAULT)
    hhT = hhT + b1_ref[...]
    hhT = hhT * jax.nn.sigmoid(hhT)
    vh = jnp.dot(wh_ref[...], hhT, preferred_element_type=jnp.float32,
                 precision=lax.Precision.DEFAULT)

    zr = z_ref[...].reshape(1, _R)
    t = jnp.dot(wz1_ref[...], zr, preferred_element_type=jnp.float32)
    t = t + bz1_ref[...]
    t = t * jax.nn.sigmoid(t)
    u = jnp.dot(wz2_ref[...], t, preferred_element_type=jnp.float32)
    u = u + bz2_ref[...]
    u = u * jax.nn.sigmoid(u)
    vz = jnp.dot(wzv_ref[...], u, preferred_element_type=jnp.float32)

    v_ref[...] = (vh + vz).reshape(1, _RROW, 128)


def _compute_v(h, z2, w1, b1c, wz1c, bz1c, wz2, bz2c, whr, wzvr):
    return pl.pallas_call(
        _tc_body,
        grid=(_NB,),
        in_specs=[
            pl.BlockSpec((_R, _H), lambda i: (i, 0)),
            pl.BlockSpec((_RROW, 128), lambda i: (i, 0)),
            pl.BlockSpec((_HO, _H), lambda i: (0, 0)),
            pl.BlockSpec((_HO, 1), lambda i: (0, 0)),
            pl.BlockSpec((16, 1), lambda i: (0, 0)),
            pl.BlockSpec((16, 1), lambda i: (0, 0)),
            pl.BlockSpec((_ZD, 16), lambda i: (0, 0)),
            pl.BlockSpec((_ZD, 1), lambda i: (0, 0)),
            pl.BlockSpec((1, _HO), lambda i: (0, 0)),
            pl.BlockSpec((1, _ZD), lambda i: (0, 0)),
        ],
        out_specs=pl.BlockSpec((1, _RROW, 128), lambda i: (i, 0, 0)),
        out_shape=jax.ShapeDtypeStruct((_NB, _RROW, 128), jnp.float32),
    )(h, z2, w1, b1c, wz1c, bz1c, wz2, bz2c, whr, wzvr)


def _sc_segment_sum(idx3, val3):
    mesh = plsc.VectorSubcoreMesh(core_axis_name="c", subcore_axis_name="s")

    @functools.partial(
        pl.kernel,
        out_type=jax.ShapeDtypeStruct((2, _ACC), jnp.float32),
        mesh=mesh,
        scratch_types=[
            pltpu.VMEM((_KW, 128), jnp.int32),
            pltpu.VMEM((_KW, 128), jnp.float32),
            pltpu.VMEM((_ACC,), jnp.float32),
            pltpu.VMEM((16, 128), jnp.float32),
            pltpu.VMEM((128,), jnp.float32),
            pltpu.VMEM_SHARED((16 * _ACC,), jnp.float32),
        ],
        compiler_params=pltpu.CompilerParams(needs_layout_passes=False),
    )
    def k(idx_hbm, val_hbm, out_hbm, idx_v, val_v, acc, rbuf, obuf, stage):
        c = lax.axis_index("c")
        s = lax.axis_index("s")
        w = s * 2 + c

        for kk in range(_ACC // 16):
            acc[pl.ds(kk * 16, 16)] = jnp.zeros((16,), jnp.float32)

        pltpu.sync_copy(idx_hbm.at[w], idx_v)
        pltpu.sync_copy(val_hbm.at[w], val_v)
        for j in range(_KW):
            for t in range(8):
                sl = pl.ds(t * 16, 16)
                plsc.addupdate_scatter(acc, [idx_v[j, sl]], val_v[j, sl])

        pltpu.sync_copy(acc, stage.at[pl.ds(s * _ACC, _ACC)])
        plsc.subcore_barrier()

        @pl.when(s < 8)
        def _reduce():
            for r in range(16):
                pltpu.sync_copy(stage.at[pl.ds(r * _ACC + s * 128, 128)],
                                rbuf.at[r])
            for c8 in range(8):
                a = rbuf[0, pl.ds(c8 * 16, 16)]
                for r in range(1, 16):
                    a = a + rbuf[r, pl.ds(c8 * 16, 16)]
                obuf[pl.ds(c8 * 16, 16)] = a
            pltpu.sync_copy(obuf, out_hbm.at[c, pl.ds(s * 128, 128)])

    return k(idx3, val3)


def kernel(h, z_coords, batch, W1, b1, Wz1, bz1, Wz2, bz2, W2, b2):
    pad = _NPAD - _N
    b1c = b1.reshape(_HO, 1)
    wz1c = Wz1.reshape(16, 1)
    bz1c = bz1.reshape(16, 1)
    bz2c = bz2.reshape(_ZD, 1)
    whr = W2[:, :_HO].reshape(1, _HO)
    wzvr = W2[:, _HO:].reshape(1, _ZD)

    z2 = jnp.concatenate(
        [z_coords.reshape(_N), jnp.zeros((pad,), jnp.float32)]
    ).reshape(_ROWS, 128)

    v = _compute_v(h, z2, W1, b1c, wz1c, bz1c, Wz2, bz2c, whr, wzvr)

    idx = batch.astype(jnp.int32)
    idx_pad = _NSEG + (jnp.arange(pad, dtype=jnp.int32) % (_ACC - _NSEG))
    idx3 = jnp.concatenate([idx, idx_pad]).reshape(_NW, _KW, 128)

    parts = _sc_segment_sum(idx3, v.reshape(_NW, _KW, 128))
    seg = parts[0, :_NSEG] + parts[1, :_NSEG]
    return (seg + b2[0]).reshape(_NSEG, 1)
